# baseline (speedup 1.0000x reference)
import functools

import jax
import jax.numpy as jnp
import numpy as np
from jax import lax
from jax.experimental import pallas as pl
from jax.experimental.pallas import tpu as pltpu
from jax.experimental.pallas import tpu_sc as plsc

NC = 2
NS = 16
NW = NC * NS

STREAM = 100
IDX_GROUP = 64


def _positional_encoding(model_size, sequence_length):
    pos = np.arange(sequence_length, dtype=np.float64)[:, None]
    i = np.arange(model_size, dtype=np.float64)[None, :]
    exponent = np.where(i % 2 == 0, i, i - 1) / model_size
    angle = pos / np.power(10000.0, exponent)
    pe = np.where(i % 2 == 0, np.sin(angle), np.cos(angle))
    return pe.astype(np.float32)


@functools.partial(jax.jit, static_argnums=(3, 4))
def _sc_embed(idx, pe, table, seq, dim):
    n_groups = idx.shape[1]
    b_per_w = n_groups * IDX_GROUP * STREAM
    chunk = seq
    streams_per_chunk = seq // STREAM
    chunks_per_group = IDX_GROUP // streams_per_chunk
    B = NW * b_per_w

    mesh = plsc.VectorSubcoreMesh(
        core_axis_name="c", subcore_axis_name="s",
        num_cores=NC, num_subcores=NS)

    @functools.partial(
        pl.kernel,
        mesh=mesh,
        compiler_params=pltpu.CompilerParams(use_tc_tiling_on_sc=False),
        out_type=jax.ShapeDtypeStruct((B, dim), jnp.float32),
        scratch_types=[
            pltpu.VMEM((IDX_GROUP, STREAM), jnp.int32),
            pltpu.VMEM((chunk, dim), jnp.float32),
            pltpu.SemaphoreType.DMA,
        ],
    )
    def k(idx_hbm, pe_hbm, table_hbm, out_hbm, idx_v, rows_v, gsem):
        wid = lax.axis_index("s") * NC + lax.axis_index("c")
        base = wid * b_per_w

        def group_body(g, _):
            pltpu.sync_copy(idx_hbm.at[wid].at[g], idx_v)

            def chunk_body(c, _):
                pltpu.sync_copy(pe_hbm, rows_v)
                for s in range(streams_per_chunk):
                    pltpu.async_copy(
                        table_hbm.at[idx_v.at[c * streams_per_chunk + s]],
                        rows_v.at[pl.ds(s * STREAM, STREAM)],
                        gsem, add=True).wait()
                out_row = base + (g * chunks_per_group + c) * chunk
                pltpu.sync_copy(rows_v, out_hbm.at[pl.ds(out_row, chunk)])
                return ()

            lax.fori_loop(0, chunks_per_group, chunk_body, ())
            return ()

        lax.fori_loop(0, n_groups, group_body, ())

    return k(idx, pe, table)


def kernel(inputs, table):
    batch, seq = inputs.shape
    vocab, dim = table.shape
    B = batch * seq
    b_per_w = B // NW
    assert B % (NW * STREAM) == 0 and seq % STREAM == 0
    n_groups = b_per_w // (IDX_GROUP * STREAM)
    assert b_per_w % (IDX_GROUP * STREAM) == 0

    idx = inputs.astype(jnp.int32).reshape(NW, n_groups, IDX_GROUP, STREAM)
    pe = jnp.asarray(_positional_encoding(dim, seq))
    out = _sc_embed(idx, pe, table, seq, dim)
    return out.reshape(batch, seq, dim)


# speedup vs baseline: 1.3552x; 1.3552x over previous
import functools

import jax
import jax.numpy as jnp
import numpy as np
from jax import lax
from jax.experimental import pallas as pl
from jax.experimental.pallas import tpu as pltpu
from jax.experimental.pallas import tpu_sc as plsc

NC = 2
NS = 16
NW = NC * NS

STREAM = 100
NBUF = 4


def _positional_encoding(model_size, sequence_length):
    pos = np.arange(sequence_length, dtype=np.float64)[:, None]
    i = np.arange(model_size, dtype=np.float64)[None, :]
    exponent = np.where(i % 2 == 0, i, i - 1) / model_size
    angle = pos / np.power(10000.0, exponent)
    pe = np.where(i % 2 == 0, np.sin(angle), np.cos(angle))
    return pe.astype(np.float32)


@functools.partial(jax.jit, static_argnums=(3, 4))
def _sc_embed(idx, pe, table, seq, dim):
    MI = idx.shape[1]
    spc = seq // STREAM
    chunk = seq
    b_per_w = MI * NBUF * chunk
    B = NW * b_per_w

    mesh = plsc.VectorSubcoreMesh(
        core_axis_name="c", subcore_axis_name="s",
        num_cores=NC, num_subcores=NS)

    @functools.partial(
        pl.kernel,
        mesh=mesh,
        compiler_params=pltpu.CompilerParams(use_tc_tiling_on_sc=False),
        out_type=jax.ShapeDtypeStruct((B, dim), jnp.float32),
        scratch_types=[
            pltpu.VMEM((2, NBUF * spc, STREAM), jnp.int32),
            pltpu.VMEM((NBUF, chunk, dim), jnp.float32),
            pltpu.VMEM_SHARED((chunk, dim), jnp.float32),
            pltpu.SemaphoreType.DMA((2,)),
            pltpu.SemaphoreType.DMA((NBUF,)),
            pltpu.SemaphoreType.DMA((NBUF,)),
            pltpu.SemaphoreType.DMA((NBUF,)),
        ],
    )
    def k(idx_hbm, pe_hbm, table_hbm, out_hbm,
          idx_v, rows_v, pe_sh, isem, psem, gsem, osem):
        sid = lax.axis_index("s")
        wid = sid * NC + lax.axis_index("c")
        base = wid * b_per_w

        @pl.when(sid == 0)
        def _():
            pltpu.sync_copy(pe_hbm, pe_sh)
        plsc.subcore_barrier()

        pltpu.async_copy(idx_hbm.at[wid].at[0], idx_v.at[0], isem.at[0])

        def mi_body(mi, _):
            ib = mi % 2
            nb = (mi + 1) % 2

            @pl.when(mi + 1 < MI)
            def _():
                pltpu.async_copy(
                    idx_hbm.at[wid].at[mi + 1], idx_v.at[nb], isem.at[nb])

            pltpu.make_async_copy(
                idx_hbm.at[wid].at[mi], idx_v.at[ib], isem.at[ib]).wait()

            for b in range(NBUF):
                @pl.when(mi >= 1)
                def _(b=b):
                    pltpu.make_async_copy(
                        rows_v.at[b], out_hbm.at[pl.ds(0, chunk)],
                        osem.at[b]).wait()
                pltpu.async_copy(pe_sh, rows_v.at[b], psem.at[b])

            for b in range(NBUF):
                pltpu.make_async_copy(pe_sh, rows_v.at[b], psem.at[b]).wait()
                for s in range(spc):
                    pltpu.async_copy(
                        table_hbm.at[idx_v.at[ib].at[b * spc + s]],
                        rows_v.at[b].at[pl.ds(s * STREAM, STREAM)],
                        gsem.at[b], add=True)

            for b in range(NBUF):
                for s in range(spc):
                    pltpu.make_async_copy(
                        table_hbm.at[idx_v.at[ib].at[b * spc + s]],
                        rows_v.at[b].at[pl.ds(s * STREAM, STREAM)],
                        gsem.at[b]).wait()
                row0 = base + (mi * NBUF + b) * chunk
                pltpu.async_copy(
                    rows_v.at[b], out_hbm.at[pl.ds(row0, chunk)], osem.at[b])
            return ()

        lax.fori_loop(0, MI, mi_body, ())

        for b in range(NBUF):
            pltpu.make_async_copy(
                rows_v.at[b], out_hbm.at[pl.ds(0, chunk)], osem.at[b]).wait()

    return k(idx, pe, table)


def kernel(inputs, table):
    batch, seq = inputs.shape
    vocab, dim = table.shape
    B = batch * seq
    b_per_w = B // NW
    spc = seq // STREAM
    assert B % NW == 0 and seq % STREAM == 0
    mi = b_per_w // (NBUF * seq)
    assert b_per_w % (NBUF * seq) == 0

    idx = inputs.astype(jnp.int32).reshape(NW, mi, NBUF * spc, STREAM)
    pe = jnp.asarray(_positional_encoding(dim, seq))
    out = _sc_embed(idx, pe, table, seq, dim)
    return out.reshape(batch, seq, dim)


# speedup vs baseline: 1.6510x; 1.2183x over previous
import functools

import jax
import jax.numpy as jnp
import numpy as np
from jax import lax
from jax.experimental import pallas as pl
from jax.experimental.pallas import tpu as pltpu
from jax.experimental.pallas import tpu_sc as plsc

NC = 2
NS = 16
NW = NC * NS

STREAM = 128
NBUF = 4


def _positional_encoding(model_size, sequence_length):
    pos = np.arange(sequence_length, dtype=np.float64)[:, None]
    i = np.arange(model_size, dtype=np.float64)[None, :]
    exponent = np.where(i % 2 == 0, i, i - 1) / model_size
    angle = pos / np.power(10000.0, exponent)
    pe = np.where(i % 2 == 0, np.sin(angle), np.cos(angle))
    return pe.astype(np.float32)


def _pe_phases(dim, seq, dim_pad):
    n_phases = seq // np.gcd(STREAM, seq)
    pe = _positional_encoding(dim, seq)
    out = np.zeros((n_phases, STREAM, dim_pad), dtype=np.float32)
    for k in range(n_phases):
        o = (STREAM * k) % seq
        rows = (o + np.arange(STREAM)) % seq
        out[k, :, :dim] = pe[rows]
    return out


@functools.partial(jax.jit, static_argnums=(3,))
def _sc_embed(idx, pe_all, table, n_streams):
    n_phases, _, dpad = pe_all.shape
    b_per_w = n_streams * STREAM
    B = NW * b_per_w

    mesh = plsc.VectorSubcoreMesh(
        core_axis_name="c", subcore_axis_name="s",
        num_cores=NC, num_subcores=NS)

    @functools.partial(
        pl.kernel,
        mesh=mesh,
        out_type=jax.ShapeDtypeStruct((B, dpad), jnp.float32),
        scratch_types=[
            pltpu.VMEM((n_streams, STREAM), jnp.int32),
            pltpu.VMEM((NBUF, STREAM, dpad), jnp.float32),
            pltpu.VMEM_SHARED((n_phases, STREAM, dpad), jnp.float32),
            pltpu.SemaphoreType.DMA((NBUF,)),
            pltpu.SemaphoreType.DMA((NBUF,)),
            pltpu.SemaphoreType.DMA((NBUF,)),
        ],
    )
    def k(idx_hbm, pe_hbm, table_hbm, out_hbm,
          idx_v, rows_v, pe_sh, psem, gsem, osem):
        tview = table_hbm
        sid = lax.axis_index("s")
        wid = sid * NC + lax.axis_index("c")
        base = wid * b_per_w

        @pl.when(sid == 0)
        def _():
            pltpu.sync_copy(pe_hbm, pe_sh)
        pltpu.sync_copy(idx_hbm.at[wid], idx_v)
        plsc.subcore_barrier()

        def mi_body(mi, _):
            for b in range(NBUF):
                j = mi * NBUF + b

                @pl.when(mi >= 1)
                def _(b=b):
                    pltpu.make_async_copy(
                        rows_v.at[b], out_hbm.at[pl.ds(0, STREAM)],
                        osem.at[b]).wait()
                pltpu.async_copy(
                    pe_sh.at[lax.rem(j, n_phases)], rows_v.at[b], psem.at[b])

            for b in range(NBUF):
                j = mi * NBUF + b
                pltpu.make_async_copy(
                    pe_sh.at[0], rows_v.at[b], psem.at[b]).wait()
                pltpu.async_copy(
                    tview.at[idx_v.at[j]], rows_v.at[b],
                    gsem.at[b], add=True)

            for b in range(NBUF):
                j = mi * NBUF + b
                pltpu.make_async_copy(
                    tview.at[idx_v.at[j]], rows_v.at[b], gsem.at[b]).wait()
                pltpu.async_copy(
                    rows_v.at[b],
                    out_hbm.at[pl.ds(base + j * STREAM, STREAM)], osem.at[b])
            return ()

        lax.fori_loop(0, n_streams // NBUF, mi_body, ())

        for b in range(NBUF):
            pltpu.make_async_copy(
                rows_v.at[b], out_hbm.at[pl.ds(0, STREAM)], osem.at[b]).wait()

    return k(idx, pe_all, table)


def kernel(inputs, table):
    batch, seq = inputs.shape
    vocab, dim = table.shape
    B = batch * seq
    dpad = 128
    assert B % (NW * STREAM) == 0
    n_streams = B // (NW * STREAM)
    assert (n_streams * STREAM) % seq == 0

    table128 = jnp.pad(table, ((0, 0), (0, dpad - dim)))
    idx = inputs.astype(jnp.int32).reshape(NW, n_streams, STREAM)
    pe_all = jnp.asarray(_pe_phases(dim, seq, dpad))
    out = _sc_embed(idx, pe_all, table128, n_streams)
    return out.reshape(batch, seq, dpad)[:, :, :dim]
